# Initial kernel scaffold; baseline (speedup 1.0000x reference)
import functools

import jax
import jax.numpy as jnp
from jax import lax
from jax.experimental import pallas as pl
from jax.experimental.pallas import tpu as pltpu
from jax.experimental.pallas import tpu_sc as plsc

NC = 2
NS = 16
LANES = 16
CHUNK = 128
N_LAYERS = 3


def _make_gcn(n, half, epad):
  n_chunks = epad // (NS * CHUNK)
  stripe = n // NS
  zrows = 625
  mrows = 125
  nz = stripe // zrows
  nm = stripe // mrows
  f32 = jnp.float32

  mesh = plsc.VectorSubcoreMesh(core_axis_name="c", subcore_axis_name="s")
  hbuf = jax.ShapeDtypeStruct((NC * n, half), f32)

  @functools.partial(
      pl.kernel,
      out_type=(hbuf, hbuf, hbuf, hbuf),
      mesh=mesh,
      scratch_types=[
          pltpu.VMEM((CHUNK,), jnp.int32),
          pltpu.VMEM((CHUNK,), jnp.int32),
          pltpu.VMEM((CHUNK,), f32),
          pltpu.VMEM((CHUNK, half), f32),
          pltpu.VMEM((625, half), f32),
          pltpu.VMEM((125, half), f32),
          pltpu.VMEM((125, half), f32),
          pltpu.VMEM((125, half), f32),
          pltpu.VMEM((125, half), f32),
          pltpu.VMEM((125, half), f32),
          pltpu.VMEM_SHARED((n, half), f32),
          pltpu.SemaphoreType.DMA,
      ],
  )
  def gcn(h0, src2, dst_h, w_h, out, h1s, h2s, h3s,
          src_v, dst_v, w_v, rows_v, zero_v, a_v, b_v, c_v, d_v, o_v,
          hsp, sem):
    cid = lax.axis_index("c")
    sid = lax.axis_index("s")
    r0 = sid * stripe
    hb = cid * n + r0
    zvec = jnp.zeros((LANES,), f32)

    def zinit(e, _):
      zero_v[e, pl.ds(0, LANES)] = zvec
      zero_v[e, pl.ds(LANES, LANES)] = zvec
      return 0
    lax.fori_loop(0, zrows, zinit, 0)

    hsrcs = [h0, h1s, h2s]
    houts = [h1s, h2s, h3s]
    for layer in range(N_LAYERS):
      hsrc = hsrcs[layer]
      hout = houts[layer]

      for z in range(nz):
        pltpu.sync_copy(zero_v, hsp.at[pl.ds(r0 + z * zrows, zrows)])
      plsc.subcore_barrier()

      def chunk_body(ct, _):
        base = (sid * n_chunks + ct) * CHUNK
        pltpu.sync_copy(src2.at[cid, pl.ds(base, CHUNK)], src_v)
        pltpu.sync_copy(dst_h.at[pl.ds(base, CHUNK)], dst_v)
        pltpu.sync_copy(w_h.at[pl.ds(base, CHUNK)], w_v)
        pltpu.async_copy(hsrc.at[src_v], rows_v, sem).wait()

        def scale(e, _):
          wv = w_v[e]
          rows_v[e, pl.ds(0, LANES)] = rows_v[e, pl.ds(0, LANES)] * wv
          rows_v[e, pl.ds(LANES, LANES)] = rows_v[e, pl.ds(LANES, LANES)] * wv
          return 0
        lax.fori_loop(0, CHUNK, scale, 0)

        pltpu.sync_copy(rows_v, hsp.at[dst_v], add=True)
        return 0
      lax.fori_loop(0, n_chunks, chunk_body, 0)
      plsc.subcore_barrier()

      pltpu.sync_copy(hsp.at[pl.ds(r0, stripe)], hout.at[pl.ds(hb, stripe)])
      plsc.subcore_barrier()

    quarter = f32(0.25)
    for z in range(nm):
      mb = hb + z * mrows
      pltpu.sync_copy(h0.at[pl.ds(mb, mrows)], a_v)
      pltpu.sync_copy(h1s.at[pl.ds(mb, mrows)], b_v)
      pltpu.sync_copy(h2s.at[pl.ds(mb, mrows)], c_v)
      pltpu.sync_copy(h3s.at[pl.ds(mb, mrows)], d_v)

      def mean_body(e, _):
        for lo in (0, LANES):
          s = pl.ds(lo, LANES)
          o_v[e, s] = (a_v[e, s] + b_v[e, s] + c_v[e, s] + d_v[e, s]) * quarter
        return 0
      lax.fori_loop(0, mrows, mean_body, 0)
      pltpu.sync_copy(o_v, out.at[pl.ds(mb, mrows)])

  return gcn


def kernel(user_emb, edge_index, edge_weight):
  n, d = user_emb.shape
  half = d // 2
  e = edge_index.shape[1]
  grp = NS * CHUNK
  epad = ((e + grp - 1) // grp) * grp

  src = edge_index[0].astype(jnp.int32)
  dst = edge_index[1].astype(jnp.int32)
  w = edge_weight.astype(jnp.float32)
  pad = epad - e
  if pad:
    src = jnp.pad(src, (0, pad))
    dst = jnp.pad(dst, (0, pad))
    w = jnp.pad(w, (0, pad))
  src2 = jnp.stack([src, src + n])
  hp = jnp.concatenate([user_emb[:, :half], user_emb[:, half:]], axis=0)

  out, _, _, _ = _make_gcn(n, half, epad)(hp, src2, dst, w)
  return jnp.concatenate([out[:n], out[n:]], axis=1)


# speedup vs baseline: 3.3384x; 3.3384x over previous
import functools

import jax
import jax.numpy as jnp
from jax import lax
from jax.experimental import pallas as pl
from jax.experimental.pallas import tpu as pltpu
from jax.experimental.pallas import tpu_sc as plsc

NC = 2
NS = 16
LANES = 16
CHUNK = 128
N_LAYERS = 3


def _make_gcn(npad, half, epad):
  n_chunks = epad // (NS * CHUNK)
  stripe = npad // NS
  zrows = 224
  mrows = 56
  nz = stripe // zrows
  nm = stripe // mrows
  assert stripe % zrows == 0 and stripe % mrows == 0
  f32 = jnp.float32

  mesh = plsc.VectorSubcoreMesh(core_axis_name="c", subcore_axis_name="s")
  hbuf = jax.ShapeDtypeStruct((NC * npad, half), f32)

  @functools.partial(
      pl.kernel,
      out_type=(hbuf, hbuf, hbuf, hbuf),
      mesh=mesh,
      compiler_params=pltpu.CompilerParams(use_tc_tiling_on_sc=False),
      scratch_types=[
          pltpu.VMEM((CHUNK,), jnp.int32),
          pltpu.VMEM((CHUNK,), jnp.int32),
          pltpu.VMEM((CHUNK,), f32),
          pltpu.VMEM((CHUNK, half), f32),
          pltpu.VMEM((zrows, half), f32),
          pltpu.VMEM((mrows, half), f32),
          pltpu.VMEM((mrows, half), f32),
          pltpu.VMEM((mrows, half), f32),
          pltpu.VMEM((mrows, half), f32),
          pltpu.VMEM((mrows, half), f32),
          pltpu.VMEM_SHARED((npad, half), f32),
          pltpu.SemaphoreType.DMA,
      ],
  )
  def gcn(h0, src2, dst_h, w_h, out, h1s, h2s, h3s,
          src_v, dst_v, w_v, rows_v, zero_v, a_v, b_v, c_v, d_v, o_v,
          hsp, sem):
    cid = lax.axis_index("c")
    sid = lax.axis_index("s")
    r0 = sid * stripe
    hb = cid * npad + r0
    zvec = jnp.zeros((LANES,), f32)

    def zinit(e, _):
      zero_v[e, pl.ds(0, LANES)] = zvec
      zero_v[e, pl.ds(LANES, LANES)] = zvec
      return 0
    lax.fori_loop(0, zrows, zinit, 0)

    hsrcs = [h0, h1s, h2s]
    houts = [h1s, h2s, h3s]
    for layer in range(N_LAYERS):
      hsrc = hsrcs[layer]
      hout = houts[layer]

      for z in range(nz):
        pltpu.sync_copy(zero_v, hsp.at[pl.ds(r0 + z * zrows, zrows)])
      plsc.subcore_barrier()

      def chunk_body(ct, _):
        base = (sid * n_chunks + ct) * CHUNK
        pltpu.sync_copy(src2.at[cid, pl.ds(base, CHUNK)], src_v)
        pltpu.sync_copy(dst_h.at[pl.ds(base, CHUNK)], dst_v)
        pltpu.sync_copy(w_h.at[pl.ds(base, CHUNK)], w_v)
        pltpu.async_copy(hsrc.at[src_v], rows_v, sem).wait()

        def scale(g, _):
          wg = w_v[pl.ds(g * LANES, LANES)]
          for k in range(LANES):
            e = g * LANES + k
            wv = wg[k]
            rows_v[e, pl.ds(0, LANES)] = rows_v[e, pl.ds(0, LANES)] * wv
            rows_v[e, pl.ds(LANES, LANES)] = rows_v[e, pl.ds(LANES, LANES)] * wv
          return 0
        lax.fori_loop(0, CHUNK // LANES, scale, 0)

        pltpu.sync_copy(rows_v, hsp.at[dst_v], add=True)
        return 0
      lax.fori_loop(0, n_chunks, chunk_body, 0)
      plsc.subcore_barrier()

      pltpu.sync_copy(hsp.at[pl.ds(r0, stripe)], hout.at[pl.ds(hb, stripe)])
      plsc.subcore_barrier()

    quarter = f32(0.25)
    for z in range(nm):
      mb = hb + z * mrows
      pltpu.sync_copy(h0.at[pl.ds(mb, mrows)], a_v)
      pltpu.sync_copy(h1s.at[pl.ds(mb, mrows)], b_v)
      pltpu.sync_copy(h2s.at[pl.ds(mb, mrows)], c_v)
      pltpu.sync_copy(h3s.at[pl.ds(mb, mrows)], d_v)

      def mean_body(e, _):
        for lo in (0, LANES):
          s = pl.ds(lo, LANES)
          o_v[e, s] = (a_v[e, s] + b_v[e, s] + c_v[e, s] + d_v[e, s]) * quarter
        return 0
      lax.fori_loop(0, mrows, mean_body, 0)
      pltpu.sync_copy(o_v, out.at[pl.ds(mb, mrows)])

  return gcn


def kernel(user_emb, edge_index, edge_weight):
  n, d = user_emb.shape
  half = d // 2
  e = edge_index.shape[1]
  grp = NS * CHUNK
  epad = ((e + grp - 1) // grp) * grp
  rgrp = NS * 64
  npad = ((n + rgrp - 1) // rgrp) * rgrp

  src = edge_index[0].astype(jnp.int32)
  dst = edge_index[1].astype(jnp.int32)
  w = edge_weight.astype(jnp.float32)
  pad = epad - e
  if pad:
    src = jnp.pad(src, (0, pad))
    dst = jnp.pad(dst, (0, pad))
    w = jnp.pad(w, (0, pad))
  src2 = jnp.stack([src, src + npad])
  hp = jnp.concatenate([user_emb[:, :half], user_emb[:, half:]], axis=0)
  hp = jnp.pad(hp.reshape(2, n, half), ((0, 0), (0, npad - n), (0, 0)))
  hp = hp.reshape(2 * npad, half)

  out, _, _, _ = _make_gcn(npad, half, epad)(hp, src2, dst, w)
  return jnp.concatenate([out[:n], out[npad:npad + n]], axis=1)


# speedup vs baseline: 6.0713x; 1.8186x over previous
import functools

import jax
import jax.numpy as jnp
from jax import lax
from jax.experimental import pallas as pl
from jax.experimental.pallas import tpu as pltpu
from jax.experimental.pallas import tpu_sc as plsc

NC = 2
NS = 16
LANES = 16
CHUNK = 128
SUP = 8
N_LAYERS = 3
ZROWS = 112
MROWS = 112


def _make_gcn(npad, half, epad):
  rows_total = epad // CHUNK
  tchunks = rows_total // NS
  npairs = tchunks // (2 * SUP)
  stripe = npad // NS
  nz = stripe // ZROWS
  nm = stripe // MROWS
  assert tchunks == npairs * 2 * SUP
  assert stripe % ZROWS == 0 and stripe % MROWS == 0
  f32 = jnp.float32

  mesh = plsc.VectorSubcoreMesh(core_axis_name="c", subcore_axis_name="s")
  hbuf = jax.ShapeDtypeStruct((NC * npad, half), f32)

  @functools.partial(
      pl.kernel,
      out_type=(hbuf, hbuf, hbuf, hbuf),
      mesh=mesh,
      compiler_params=pltpu.CompilerParams(use_tc_tiling_on_sc=False),
      scratch_types=[
          pltpu.VMEM((CHUNK, half), f32),
          pltpu.VMEM((CHUNK, half), f32),
          pltpu.VMEM((SUP, CHUNK), jnp.int32),
          pltpu.VMEM((SUP, CHUNK), jnp.int32),
          pltpu.VMEM((SUP, CHUNK), jnp.int32),
          pltpu.VMEM((SUP, CHUNK), jnp.int32),
          pltpu.VMEM((SUP, CHUNK), f32),
          pltpu.VMEM((SUP, CHUNK), f32),
          pltpu.VMEM((MROWS, half), f32),
          pltpu.VMEM((MROWS, half), f32),
          pltpu.VMEM((MROWS, half), f32),
          pltpu.VMEM((MROWS, half), f32),
          pltpu.VMEM_SHARED((npad, half), f32),
          pltpu.SemaphoreType.DMA,
          pltpu.SemaphoreType.DMA,
          pltpu.SemaphoreType.DMA,
      ],
  )
  def gcn(h0, src3, dst3, w3, out, h1s, h2s, h3s,
          rows0, rows1, sb0, sb1, db0, db1, wb0, wb1,
          ma, mb, mc, md, hsp, gsem, isem, zsem):
    cid = lax.axis_index("c")
    sid = lax.axis_index("s")
    r0 = sid * stripe
    hb = cid * npad + r0
    tb = sid * tchunks
    rows = (rows0, rows1)
    sbs, dbs, wbs = (sb0, sb1), (db0, db1), (wb0, wb1)
    zvec = jnp.zeros((LANES,), f32)

    def idx_load(srow, p, sync):
      copy = pltpu.sync_copy if sync else (
          lambda a, b: pltpu.async_copy(a, b, isem))
      copy(src3.at[cid, pl.ds(srow, SUP)], sbs[p])
      copy(dst3.at[pl.ds(srow, SUP)], dbs[p])
      copy(w3.at[pl.ds(srow, SUP)], wbs[p])

    def idx_drain(p):
      pltpu.make_async_copy(src3.at[cid, pl.ds(tb, SUP)], sbs[p], isem).wait()
      pltpu.make_async_copy(dst3.at[pl.ds(tb, SUP)], dbs[p], isem).wait()
      pltpu.make_async_copy(w3.at[pl.ds(tb, SUP)], wbs[p], isem).wait()

    def gather_start(hsrc, p, j, b):
      pltpu.async_copy(hsrc.at[sbs[p].at[j]], rows[b], gsem)

    def gather_wait(hsrc, p, j, b):
      pltpu.make_async_copy(hsrc.at[sbs[p].at[j]], rows[b], gsem).wait()

    def scale_scatter(p, j, b):
      def scale_g(g, _):
        wg = wbs[p][j, pl.ds(g * LANES, LANES)]
        for k in range(LANES):
          e = g * LANES + k
          wv = wg[k]
          rows[b][e, pl.ds(0, LANES)] = rows[b][e, pl.ds(0, LANES)] * wv
          rows[b][e, pl.ds(LANES, LANES)] = (
              rows[b][e, pl.ds(LANES, LANES)] * wv)
        return 0
      lax.fori_loop(0, CHUNK // LANES, scale_g, 0)
      pltpu.sync_copy(rows[b], hsp.at[dbs[p].at[j]], add=True)

    hsrcs = [h0, h1s, h2s]
    houts = [h1s, h2s, h3s]
    for layer in range(N_LAYERS):
      hsrc = hsrcs[layer]
      hout = houts[layer]

      idx_load(tb, 0, True)
      idx_load(tb + SUP, 1, False)

      def zinit(e, _):
        rows1[e, pl.ds(0, LANES)] = zvec
        rows1[e, pl.ds(LANES, LANES)] = zvec
        return 0
      lax.fori_loop(0, ZROWS, zinit, 0)
      for z in range(nz):
        pltpu.async_copy(rows1.at[pl.ds(0, ZROWS)],
                         hsp.at[pl.ds(r0 + z * ZROWS, ZROWS)], zsem)
      gather_start(hsrc, 0, 0, 0)
      for z in range(nz):
        pltpu.make_async_copy(rows1.at[pl.ds(0, ZROWS)],
                              hsp.at[pl.ds(r0, ZROWS)], zsem).wait()
      plsc.subcore_barrier()

      def pair_body(t, _):
        not_last = t < npairs - 1
        for half_id in range(2):
          p = half_id
          q = 1 - half_id
          srow = tb + (2 * t + half_id) * SUP
          for j in range(SUP):
            b = j % 2
            gather_wait(hsrc, p, j, b)
            if j < SUP - 1:
              gather_start(hsrc, p, j + 1, 1 - b)
              scale_scatter(p, j, b)
            else:
              scale_scatter(p, j, b)
              if half_id == 0:
                idx_drain(q)

                @pl.when(not_last)
                def _():
                  idx_load(srow + 2 * SUP, p, False)
                gather_start(hsrc, q, 0, 0)
              else:
                @pl.when(not_last)
                def _():
                  idx_drain(q)
                  idx_load(srow + 2 * SUP, p, False)
                  gather_start(hsrc, q, 0, 0)
        return 0
      lax.fori_loop(0, npairs, pair_body, 0)
      plsc.subcore_barrier()

      pltpu.sync_copy(hsp.at[pl.ds(r0, stripe)], hout.at[pl.ds(hb, stripe)])
      plsc.subcore_barrier()

    quarter = f32(0.25)

    def mean_chunk(z, _):
      mbase = hb + z * MROWS
      pltpu.async_copy(h0.at[pl.ds(mbase, MROWS)], ma, zsem)
      pltpu.async_copy(h1s.at[pl.ds(mbase, MROWS)], mb, zsem)
      pltpu.async_copy(h2s.at[pl.ds(mbase, MROWS)], mc, zsem)
      pltpu.async_copy(h3s.at[pl.ds(mbase, MROWS)], md, zsem)
      for _buf in (ma, mb, mc, md):
        pltpu.make_async_copy(h0.at[pl.ds(hb, MROWS)], _buf, zsem).wait()

      def mean_body(e, _):
        for lo in (0, LANES):
          s = pl.ds(lo, LANES)
          ma[e, s] = (ma[e, s] + mb[e, s] + mc[e, s] + md[e, s]) * quarter
        return 0
      lax.fori_loop(0, MROWS, mean_body, 0)
      pltpu.sync_copy(ma, out.at[pl.ds(mbase, MROWS)])
      return 0
    lax.fori_loop(0, nm, mean_chunk, 0)

  return gcn


def kernel(user_emb, edge_index, edge_weight):
  n, d = user_emb.shape
  half = d // 2
  e = edge_index.shape[1]
  grp = NS * CHUNK * SUP * 2
  epad = ((e + grp - 1) // grp) * grp
  rgrp = NS * 64
  npad = ((n + rgrp - 1) // rgrp) * rgrp

  src = edge_index[0].astype(jnp.int32)
  dst = edge_index[1].astype(jnp.int32)
  w = edge_weight.astype(jnp.float32)
  pad = epad - e
  if pad:
    src = jnp.pad(src, (0, pad))
    dst = jnp.pad(dst, (0, pad))
    w = jnp.pad(w, (0, pad))
  src3 = jnp.stack([src, src + npad]).reshape(2, epad // CHUNK, CHUNK)
  dst3 = dst.reshape(epad // CHUNK, CHUNK)
  w3 = w.reshape(epad // CHUNK, CHUNK)
  hp = jnp.concatenate([user_emb[:, :half], user_emb[:, half:]], axis=0)
  hp = jnp.pad(hp.reshape(2, n, half), ((0, 0), (0, npad - n), (0, 0)))
  hp = hp.reshape(2 * npad, half)

  out, _, _, _ = _make_gcn(npad, half, epad)(hp, src3, dst3, w3)
  return jnp.concatenate([out[:n], out[npad:npad + n]], axis=1)
